# baseline (speedup 1.0000x reference)
import functools

import jax
import jax.numpy as jnp
from jax import lax
from jax.experimental import pallas as pl
from jax.experimental.pallas import tpu as pltpu
from jax.experimental.pallas import tpu_sc as plsc

N = 10000
N_PAD = 10240
NC = 2
NS = 16
CHUNK = 128
CPT = 81
E_PAD = NC * NS * CPT * CHUNK
RPS = N_PAD // NS
NEGH = -5e29
F32 = jnp.float32


def _sc_mesh():
    return plsc.VectorSubcoreMesh(
        core_axis_name="c", subcore_axis_name="s", num_cores=NC, num_subcores=NS)


def _make_edge(heads8):
    def body(src_hbm, dst_hbm, big_hbm, adst_hbm, m16_hbm, z80_hbm,
             opart_hbm,
             src_v, dst_v, gs0, gd0, gs1, gd1, cm0, cm1, m16_v, acc_sp,
             sem_a, sem_b, sem_ca, sem_cb):
        c = lax.axis_index("c")
        s = lax.axis_index("s")
        wid = c * NS + s
        pltpu.sync_copy(z80_hbm.at[pl.ds(s * RPS, RPS)],
                        acc_sp.at[pl.ds(s * RPS, RPS)])
        pltpu.sync_copy(m16_hbm, m16_v)
        pltpu.sync_copy(src_hbm.at[wid], src_v)
        pltpu.sync_copy(dst_hbm.at[wid], dst_v)
        plsc.subcore_barrier()
        m16 = m16_v[...]
        iota16 = lax.broadcasted_iota(jnp.int32, (16,), 0)

        def issue(j, gs, gd, sem):
            pltpu.async_copy(big_hbm.at[src_v.at[j]], gs, sem)
            pltpu.async_copy(adst_hbm.at[dst_v.at[j]], gd, sem)

        def wait(gs, gd, sem):
            pltpu.make_async_copy(big_hbm.at[src_v.at[0]], gs, sem).wait()
            pltpu.make_async_copy(adst_hbm.at[dst_v.at[0]], gd, sem).wait()

        def compute(j, gs, gd, cm, sem_c):
            def edge_body(k, carry2):
                e = gs[k, pl.ds(0, 16)] + gd[k]
                e = jnp.where(e > 0, e, 0.2 * e)
                cm[k, pl.ds(64, 16)] = jnp.exp(e - m16)
                ksplat = jnp.full((16,), 0, jnp.int32) + k
                for v in range(4):
                    if heads8:
                        pat = 64 + 2 * v + jnp.where(iota16 >= 8, 1, 0)
                    else:
                        pat = 64 + iota16 * 0
                    av = plsc.load_gather(cm, [ksplat, pat])
                    cm[k, pl.ds(16 * v, 16)] = (
                        gs[k, pl.ds(16 + 16 * v, 16)] * av)
                return carry2

            lax.fori_loop(0, CHUNK, edge_body, 0, unroll=8)
            pltpu.async_copy(cm, acc_sp.at[dst_v.at[j]], sem_c, add=True)

        def drain(cm, sem_c):
            pltpu.make_async_copy(cm, acc_sp.at[dst_v.at[0]], sem_c).wait()

        issue(0, gs0, gd0, sem_a)

        def pair_body(t, carry):
            j0 = 2 * t
            issue(j0 + 1, gs1, gd1, sem_b)
            wait(gs0, gd0, sem_a)

            @pl.when(j0 >= 2)
            def _():
                drain(cm0, sem_ca)

            compute(j0, gs0, gd0, cm0, sem_ca)

            @pl.when(j0 + 2 < CPT)
            def _():
                issue(j0 + 2, gs0, gd0, sem_a)

            wait(gs1, gd1, sem_b)

            @pl.when(j0 >= 2)
            def _():
                drain(cm1, sem_cb)

            compute(j0 + 1, gs1, gd1, cm1, sem_cb)
            return carry

        lax.fori_loop(0, CPT // 2, pair_body, 0)
        wait(gs0, gd0, sem_a)
        drain(cm0, sem_ca)
        compute(CPT - 1, gs0, gd0, cm0, sem_ca)
        drain(cm0, sem_ca)
        drain(cm1, sem_cb)
        plsc.subcore_barrier()
        pltpu.sync_copy(acc_sp.at[pl.ds(s * RPS, RPS)],
                        opart_hbm.at[c, pl.ds(s * RPS, RPS)])

    return functools.partial(
        pl.kernel,
        out_type=[jax.ShapeDtypeStruct((NC, N_PAD, 80), F32)],
        mesh=_sc_mesh(),
        compiler_params=pltpu.CompilerParams(use_tc_tiling_on_sc=False,
                                             needs_layout_passes=False),
        scratch_types=[
            pltpu.VMEM((CPT, CHUNK), jnp.int32),
            pltpu.VMEM((CPT, CHUNK), jnp.int32),
            pltpu.VMEM((CHUNK, 80), F32),
            pltpu.VMEM((CHUNK, 16), F32),
            pltpu.VMEM((CHUNK, 80), F32),
            pltpu.VMEM((CHUNK, 16), F32),
            pltpu.VMEM((CHUNK, 80), F32),
            pltpu.VMEM((CHUNK, 80), F32),
            pltpu.VMEM((16,), F32),
            pltpu.VMEM_SHARED((N_PAD, 80), F32),
            pltpu.SemaphoreType.DMA,
            pltpu.SemaphoreType.DMA,
            pltpu.SemaphoreType.DMA,
            pltpu.SemaphoreType.DMA,
        ],
    )(body)


_BLK = 512
_NBLK = N_PAD // _BLK


def _tables1_body(x_ref, w_ref, bs_ref, bd_ref, pv_ref,
                  big_ref, as_ref, ad_ref):
    h = jnp.dot(x_ref[...], w_ref[...], preferred_element_type=F32)
    asrc = jnp.dot(h, bs_ref[...], preferred_element_type=F32) + pv_ref[...]
    big_ref[...] = jnp.concatenate([asrc, h], axis=1)
    as_ref[...] = asrc
    ad_ref[...] = jnp.dot(h, bd_ref[...], preferred_element_type=F32) + pv_ref[...]


def _tables_next_body(p0_ref, p1_ref, ex_ref, b_ref, w_ref, bs_ref, bd_ref,
                      pv_ref, big_ref, as_ref, ad_ref):
    p = p0_ref[...] + p1_ref[...]
    dexp = jnp.dot(p[:, 64:80], ex_ref[...], preferred_element_type=F32) + 1e-16
    xx = jax.nn.relu(p[:, 0:64] / dexp + b_ref[...])
    h = jnp.dot(xx, w_ref[...], preferred_element_type=F32)
    asrc = jnp.dot(h, bs_ref[...], preferred_element_type=F32) + pv_ref[...]
    big_ref[...] = jnp.concatenate([asrc, h], axis=1)
    as_ref[...] = asrc
    ad_ref[...] = jnp.dot(h, bd_ref[...], preferred_element_type=F32) + pv_ref[...]


def _tables3_body(p0_ref, p1_ref, ex_ref, b_ref, w_ref, as3_ref, ad3_ref,
                  pv_ref, big_ref, as_ref, ad_ref):
    p = p0_ref[...] + p1_ref[...]
    dexp = jnp.dot(p[:, 64:80], ex_ref[...], preferred_element_type=F32) + 1e-16
    xx = jax.nn.relu(p[:, 0:64] / dexp + b_ref[...])
    hw = jnp.dot(xx, w_ref[...], preferred_element_type=F32)
    asrc = jnp.dot(hw, as3_ref[...], preferred_element_type=F32) + pv_ref[...]
    big_ref[...] = jnp.concatenate([asrc, xx], axis=1)
    as_ref[...] = asrc
    ad_ref[...] = jnp.dot(hw, ad3_ref[...], preferred_element_type=F32) + pv_ref[...]


def _m16_body(as_ref, ad_ref, o_ref):
    m = jnp.max(as_ref[...], axis=0) + jnp.max(ad_ref[...], axis=0)
    m = jnp.where(m > 0, m, 0.2 * m)
    o_ref[...] = jnp.broadcast_to(m[None, :], (8, 16))


def _final_body(p0_ref, p1_ref, ex_ref, b3e_ref, b3o_ref, w3e_ref, w3o_ref,
                s0_ref, s1_ref):
    p = p0_ref[...] + p1_ref[...]
    dexp = jnp.dot(p[:, 64:80], ex_ref[...], preferred_element_type=F32) + 1e-16
    agg = p[:, 0:64] / dexp
    z0 = jax.nn.relu(jnp.dot(agg, w3e_ref[...], preferred_element_type=F32)
                     + b3e_ref[...])
    z1 = jax.nn.relu(jnp.dot(agg, w3o_ref[...], preferred_element_type=F32)
                     + b3o_ref[...])
    m = jnp.maximum(z0, z1)
    e0 = jnp.exp(z0 - m)
    e1 = jnp.exp(z1 - m)
    t = e0 + e1
    s0_ref[...] = e0 / t
    s1_ref[...] = e1 / t


def _row_spec(cols):
    return pl.BlockSpec((_BLK, cols), lambda i: (i, 0))


def _full_spec(shape):
    return pl.BlockSpec(shape, lambda i: tuple(0 for _ in shape))


_TBL_OUT = [jax.ShapeDtypeStruct((N_PAD, 80), F32),
            jax.ShapeDtypeStruct((N_PAD, 16), F32),
            jax.ShapeDtypeStruct((N_PAD, 16), F32)]
_TBL_OUT_SPECS = [_row_spec(80), _row_spec(16), _row_spec(16)]


def _tables1(xp, W1, Bs, Bd, pv):
    return pl.pallas_call(
        _tables1_body,
        grid=(_NBLK,),
        in_specs=[_row_spec(128), _full_spec((128, 64)), _full_spec((64, 16)),
                  _full_spec((64, 16)), _full_spec((1, 16))],
        out_specs=_TBL_OUT_SPECS,
        out_shape=_TBL_OUT,
    )(xp, W1, Bs, Bd, pv)


def _tables_next(p0, p1, ex, b, W, Bs, Bd, pv):
    return pl.pallas_call(
        _tables_next_body,
        grid=(_NBLK,),
        in_specs=[_row_spec(80), _row_spec(80), _full_spec((16, 64)),
                  _full_spec((1, 64)), _full_spec((64, 64)),
                  _full_spec((64, 16)), _full_spec((64, 16)),
                  _full_spec((1, 16))],
        out_specs=_TBL_OUT_SPECS,
        out_shape=_TBL_OUT,
    )(p0, p1, ex, b, W, Bs, Bd, pv)


def _tables3(p0, p1, ex, b, W3, A_s, A_d, pv):
    return pl.pallas_call(
        _tables3_body,
        grid=(_NBLK,),
        in_specs=[_row_spec(80), _row_spec(80), _full_spec((16, 64)),
                  _full_spec((1, 64)), _full_spec((64, 242)),
                  _full_spec((242, 16)), _full_spec((242, 16)),
                  _full_spec((1, 16))],
        out_specs=_TBL_OUT_SPECS,
        out_shape=_TBL_OUT,
    )(p0, p1, ex, b, W3, A_s, A_d, pv)


def _m16(asrc, adst):
    out = pl.pallas_call(
        _m16_body,
        out_shape=jax.ShapeDtypeStruct((8, 16), F32),
    )(asrc, adst)
    return out[0]


def _final(p0, p1, ex, b3e, b3o, W3e, W3o):
    return pl.pallas_call(
        _final_body,
        grid=(_NBLK,),
        in_specs=[_row_spec(80), _row_spec(80), _full_spec((16, 64)),
                  _full_spec((1, 121)), _full_spec((1, 121)),
                  _full_spec((64, 121)), _full_spec((64, 121))],
        out_specs=[_row_spec(121), _row_spec(121)],
        out_shape=[jax.ShapeDtypeStruct((N_PAD, 121), F32),
                   jax.ShapeDtypeStruct((N_PAD, 121), F32)],
    )(p0, p1, ex, b3e, b3o, W3e, W3o)


def _blockdiag(a):
    eye8 = jnp.eye(8, dtype=F32)
    B = (a.astype(F32)[:, :, None] * eye8[:, None, :]).reshape(64, 8)
    return jnp.pad(B, ((0, 0), (0, 8)))


def kernel(x, edge_index, W1, a_src1, a_dst1, b1, W2, a_src2, a_dst2, b2,
           W3, a_src3, a_dst3, b3):
    x = x.astype(F32)
    loop = jnp.arange(N, dtype=jnp.int32)
    src = jnp.concatenate([edge_index[0].astype(jnp.int32), loop])
    dst = jnp.concatenate([edge_index[1].astype(jnp.int32), loop])
    pad_e = E_PAD - src.shape[0]
    src = jnp.concatenate([src, jnp.full((pad_e,), N, jnp.int32)])
    dst = jnp.concatenate([dst, jnp.full((pad_e,), N, jnp.int32)])
    src = src.reshape(NC * NS, CPT, CHUNK)
    dst = dst.reshape(NC * NS, CPT, CHUNK)

    xp = jnp.pad(x, ((0, N_PAD - N), (0, 0)))
    Bs1, Bd1 = _blockdiag(a_src1), _blockdiag(a_dst1)
    Bs2, Bd2 = _blockdiag(a_src2), _blockdiag(a_dst2)
    A_s = jnp.pad(jnp.tile(a_src3.astype(F32).reshape(242, 1), (1, 8)),
                  ((0, 0), (0, 8)))
    A_d = jnp.pad(jnp.tile(a_dst3.astype(F32).reshape(242, 1), (1, 8)),
                  ((0, 0), (0, 8)))
    pv = jnp.concatenate([jnp.zeros((8,), F32),
                          jnp.full((8,), NEGH, F32)]).reshape(1, 16)
    ex8 = jnp.pad(jnp.kron(jnp.eye(8, dtype=F32), jnp.ones((1, 8), F32)),
                  ((0, 8), (0, 0)))
    ex1 = jnp.zeros((16, 64), F32).at[0, :].set(1.0)
    z80 = jnp.zeros((N_PAD, 80), F32)
    W3e = W3.astype(F32)[:, 0::2]
    W3o = W3.astype(F32)[:, 1::2]
    b3e = b3.astype(F32)[0::2].reshape(1, 121)
    b3o = b3.astype(F32)[1::2].reshape(1, 121)
    b1r = b1.astype(F32).reshape(1, 64)
    b2r = b2.astype(F32).reshape(1, 64)

    edge_h8 = _make_edge(True)
    edge_h1 = _make_edge(False)

    big1, as1, ad1 = _tables1(xp, W1.astype(F32), Bs1, Bd1, pv)
    m1 = _m16(as1, ad1)
    op1, = edge_h8(src, dst, big1, ad1, m1, z80)
    big2, as2, ad2 = _tables_next(op1[0], op1[1], ex8, b1r, W2.astype(F32),
                                  Bs2, Bd2, pv)
    m2 = _m16(as2, ad2)
    op2, = edge_h8(src, dst, big2, ad2, m2, z80)
    big3, as3, ad3 = _tables3(op2[0], op2[1], ex8, b2r, W3.astype(F32),
                              A_s, A_d, pv)
    m3 = _m16(as3, ad3)
    op3, = edge_h1(src, dst, big3, ad3, m3, z80)
    s0, s1 = _final(op3[0], op3[1], ex1, b3e, b3o, W3e, W3o)
    return jnp.stack([s0[:N], s1[:N]], axis=-1)


# speedup vs baseline: 1.0509x; 1.0509x over previous
import functools

import jax
import jax.numpy as jnp
from jax import lax
from jax.experimental import pallas as pl
from jax.experimental.pallas import tpu as pltpu
from jax.experimental.pallas import tpu_sc as plsc

N = 10000
N_PAD = 10240
NC = 2
NS = 16
CHUNK = 128
CPT = 81
E_PAD = NC * NS * CPT * CHUNK
RPS = N_PAD // NS
NEGH = -5e29
F32 = jnp.float32


def _sc_mesh():
    return plsc.VectorSubcoreMesh(
        core_axis_name="c", subcore_axis_name="s", num_cores=NC, num_subcores=NS)


def _make_edge(heads8):
    def body(src_hbm, dst_hbm, big_hbm, adst_hbm, mm_hbm, z80_hbm,
             opart_hbm,
             src_v, dst_v, gs0, gd0, gs1, gd1, cm0, cm1, mm_v, acc_sp,
             sem_a, sem_b, sem_ca, sem_cb):
        c = lax.axis_index("c")
        s = lax.axis_index("s")
        wid = c * NS + s
        pltpu.sync_copy(z80_hbm.at[pl.ds(s * RPS, RPS)],
                        acc_sp.at[pl.ds(s * RPS, RPS)])
        pltpu.sync_copy(mm_hbm, mm_v)
        pltpu.sync_copy(src_hbm.at[wid], src_v)
        pltpu.sync_copy(dst_hbm.at[wid], dst_v)
        plsc.subcore_barrier()
        m16 = mm_v[0] + mm_v[4]
        m16 = jnp.where(m16 > 0, m16, 0.2 * m16)
        iota16 = lax.broadcasted_iota(jnp.int32, (16,), 0)

        def issue(j, gs, gd, sem):
            pltpu.async_copy(big_hbm.at[src_v.at[j]], gs, sem)
            pltpu.async_copy(adst_hbm.at[dst_v.at[j]], gd, sem)

        def wait(gs, gd, sem):
            pltpu.make_async_copy(big_hbm.at[src_v.at[0]], gs, sem).wait()
            pltpu.make_async_copy(adst_hbm.at[dst_v.at[0]], gd, sem).wait()

        def compute(j, gs, gd, cm, sem_c):
            def edge_body(k, carry2):
                e = gs[k, pl.ds(0, 16)] + gd[k]
                e = jnp.where(e > 0, e, 0.2 * e)
                cm[k, pl.ds(64, 16)] = jnp.exp(e - m16)
                ksplat = jnp.full((16,), 0, jnp.int32) + k
                for v in range(4):
                    if heads8:
                        pat = 64 + 2 * v + jnp.where(iota16 >= 8, 1, 0)
                    else:
                        pat = 64 + iota16 * 0
                    av = plsc.load_gather(cm, [ksplat, pat])
                    cm[k, pl.ds(16 * v, 16)] = (
                        gs[k, pl.ds(16 + 16 * v, 16)] * av)
                return carry2

            lax.fori_loop(0, CHUNK, edge_body, 0, unroll=4)
            pltpu.async_copy(cm, acc_sp.at[dst_v.at[j]], sem_c, add=True)

        def drain(cm, sem_c):
            pltpu.make_async_copy(cm, acc_sp.at[dst_v.at[0]], sem_c).wait()

        issue(0, gs0, gd0, sem_a)

        def pair_body(t, carry):
            j0 = 2 * t
            issue(j0 + 1, gs1, gd1, sem_b)
            wait(gs0, gd0, sem_a)

            @pl.when(j0 >= 2)
            def _():
                drain(cm0, sem_ca)

            compute(j0, gs0, gd0, cm0, sem_ca)

            @pl.when(j0 + 2 < CPT)
            def _():
                issue(j0 + 2, gs0, gd0, sem_a)

            wait(gs1, gd1, sem_b)

            @pl.when(j0 >= 2)
            def _():
                drain(cm1, sem_cb)

            compute(j0 + 1, gs1, gd1, cm1, sem_cb)
            return carry

        lax.fori_loop(0, CPT // 2, pair_body, 0)
        wait(gs0, gd0, sem_a)
        drain(cm0, sem_ca)
        compute(CPT - 1, gs0, gd0, cm0, sem_ca)
        drain(cm0, sem_ca)
        drain(cm1, sem_cb)
        plsc.subcore_barrier()
        pltpu.sync_copy(acc_sp.at[pl.ds(s * RPS, RPS)],
                        opart_hbm.at[c, pl.ds(s * RPS, RPS)])

    return functools.partial(
        pl.kernel,
        out_type=[jax.ShapeDtypeStruct((NC, N_PAD, 80), F32)],
        mesh=_sc_mesh(),
        compiler_params=pltpu.CompilerParams(use_tc_tiling_on_sc=False,
                                             needs_layout_passes=False),
        scratch_types=[
            pltpu.VMEM((CPT, CHUNK), jnp.int32),
            pltpu.VMEM((CPT, CHUNK), jnp.int32),
            pltpu.VMEM((CHUNK, 80), F32),
            pltpu.VMEM((CHUNK, 16), F32),
            pltpu.VMEM((CHUNK, 80), F32),
            pltpu.VMEM((CHUNK, 16), F32),
            pltpu.VMEM((CHUNK, 80), F32),
            pltpu.VMEM((CHUNK, 80), F32),
            pltpu.VMEM((8, 16), F32),
            pltpu.VMEM_SHARED((N_PAD, 80), F32),
            pltpu.SemaphoreType.DMA,
            pltpu.SemaphoreType.DMA,
            pltpu.SemaphoreType.DMA,
            pltpu.SemaphoreType.DMA,
        ],
    )(body)


_BLK = 512
_NBLK = N_PAD // _BLK


def _mm_update(asrc, adst, mm_ref):
    i = pl.program_id(0)
    upd = jnp.concatenate(
        [jnp.broadcast_to(jnp.max(asrc, axis=0)[None, :], (4, 16)),
         jnp.broadcast_to(jnp.max(adst, axis=0)[None, :], (4, 16))], axis=0)

    @pl.when(i == 0)
    def _():
        mm_ref[...] = upd

    @pl.when(i > 0)
    def _():
        mm_ref[...] = jnp.maximum(mm_ref[...], upd)


def _tables1_body(x_ref, w_ref, bs_ref, bd_ref, pv_ref,
                  big_ref, ad_ref, mm_ref):
    h = jnp.dot(x_ref[...], w_ref[...], preferred_element_type=F32)
    asrc = jnp.dot(h, bs_ref[...], preferred_element_type=F32) + pv_ref[...]
    big_ref[...] = jnp.concatenate([asrc, h], axis=1)
    adst = jnp.dot(h, bd_ref[...], preferred_element_type=F32) + pv_ref[...]
    ad_ref[...] = adst
    _mm_update(asrc, adst, mm_ref)


def _tables_next_body(p0_ref, p1_ref, ex_ref, b_ref, w_ref, bs_ref, bd_ref,
                      pv_ref, big_ref, ad_ref, mm_ref):
    p = p0_ref[...] + p1_ref[...]
    dexp = jnp.dot(p[:, 64:80], ex_ref[...], preferred_element_type=F32) + 1e-16
    xx = jax.nn.relu(p[:, 0:64] / dexp + b_ref[...])
    h = jnp.dot(xx, w_ref[...], preferred_element_type=F32)
    asrc = jnp.dot(h, bs_ref[...], preferred_element_type=F32) + pv_ref[...]
    big_ref[...] = jnp.concatenate([asrc, h], axis=1)
    adst = jnp.dot(h, bd_ref[...], preferred_element_type=F32) + pv_ref[...]
    ad_ref[...] = adst
    _mm_update(asrc, adst, mm_ref)


def _tables3_body(p0_ref, p1_ref, ex_ref, b_ref, w_ref, as3_ref, ad3_ref,
                  pv_ref, big_ref, ad_ref, mm_ref):
    p = p0_ref[...] + p1_ref[...]
    dexp = jnp.dot(p[:, 64:80], ex_ref[...], preferred_element_type=F32) + 1e-16
    xx = jax.nn.relu(p[:, 0:64] / dexp + b_ref[...])
    hw = jnp.dot(xx, w_ref[...], preferred_element_type=F32)
    asrc = jnp.dot(hw, as3_ref[...], preferred_element_type=F32) + pv_ref[...]
    big_ref[...] = jnp.concatenate([asrc, xx], axis=1)
    adst = jnp.dot(hw, ad3_ref[...], preferred_element_type=F32) + pv_ref[...]
    ad_ref[...] = adst
    _mm_update(asrc, adst, mm_ref)


def _final_body(p0_ref, p1_ref, ex_ref, b3e_ref, b3o_ref, w3e_ref, w3o_ref,
                s0_ref, s1_ref):
    p = p0_ref[...] + p1_ref[...]
    dexp = jnp.dot(p[:, 64:80], ex_ref[...], preferred_element_type=F32) + 1e-16
    agg = p[:, 0:64] / dexp
    z0 = jax.nn.relu(jnp.dot(agg, w3e_ref[...], preferred_element_type=F32)
                     + b3e_ref[...])
    z1 = jax.nn.relu(jnp.dot(agg, w3o_ref[...], preferred_element_type=F32)
                     + b3o_ref[...])
    m = jnp.maximum(z0, z1)
    e0 = jnp.exp(z0 - m)
    e1 = jnp.exp(z1 - m)
    t = e0 + e1
    s0_ref[...] = e0 / t
    s1_ref[...] = e1 / t


def _row_spec(cols):
    return pl.BlockSpec((_BLK, cols), lambda i: (i, 0))


def _full_spec(shape):
    return pl.BlockSpec(shape, lambda i: tuple(0 for _ in shape))


_TBL_OUT = [jax.ShapeDtypeStruct((N_PAD, 80), F32),
            jax.ShapeDtypeStruct((N_PAD, 16), F32),
            jax.ShapeDtypeStruct((8, 16), F32)]
_TBL_OUT_SPECS = [_row_spec(80), _row_spec(16),
                  pl.BlockSpec((8, 16), lambda i: (0, 0))]


def _tables1(xp, W1, Bs, Bd, pv):
    return pl.pallas_call(
        _tables1_body,
        grid=(_NBLK,),
        in_specs=[_row_spec(128), _full_spec((128, 64)), _full_spec((64, 16)),
                  _full_spec((64, 16)), _full_spec((1, 16))],
        out_specs=_TBL_OUT_SPECS,
        out_shape=_TBL_OUT,
    )(xp, W1, Bs, Bd, pv)


def _tables_next(p0, p1, ex, b, W, Bs, Bd, pv):
    return pl.pallas_call(
        _tables_next_body,
        grid=(_NBLK,),
        in_specs=[_row_spec(80), _row_spec(80), _full_spec((16, 64)),
                  _full_spec((1, 64)), _full_spec((64, 64)),
                  _full_spec((64, 16)), _full_spec((64, 16)),
                  _full_spec((1, 16))],
        out_specs=_TBL_OUT_SPECS,
        out_shape=_TBL_OUT,
    )(p0, p1, ex, b, W, Bs, Bd, pv)


def _tables3(p0, p1, ex, b, W3, A_s, A_d, pv):
    return pl.pallas_call(
        _tables3_body,
        grid=(_NBLK,),
        in_specs=[_row_spec(80), _row_spec(80), _full_spec((16, 64)),
                  _full_spec((1, 64)), _full_spec((64, 242)),
                  _full_spec((242, 16)), _full_spec((242, 16)),
                  _full_spec((1, 16))],
        out_specs=_TBL_OUT_SPECS,
        out_shape=_TBL_OUT,
    )(p0, p1, ex, b, W3, A_s, A_d, pv)


def _final(p0, p1, ex, b3e, b3o, W3e, W3o):
    return pl.pallas_call(
        _final_body,
        grid=(_NBLK,),
        in_specs=[_row_spec(80), _row_spec(80), _full_spec((16, 64)),
                  _full_spec((1, 121)), _full_spec((1, 121)),
                  _full_spec((64, 121)), _full_spec((64, 121))],
        out_specs=[_row_spec(121), _row_spec(121)],
        out_shape=[jax.ShapeDtypeStruct((N_PAD, 121), F32),
                   jax.ShapeDtypeStruct((N_PAD, 121), F32)],
    )(p0, p1, ex, b3e, b3o, W3e, W3o)


def _blockdiag(a):
    eye8 = jnp.eye(8, dtype=F32)
    B = (a.astype(F32)[:, :, None] * eye8[:, None, :]).reshape(64, 8)
    return jnp.pad(B, ((0, 0), (0, 8)))


def kernel(x, edge_index, W1, a_src1, a_dst1, b1, W2, a_src2, a_dst2, b2,
           W3, a_src3, a_dst3, b3):
    x = x.astype(F32)
    loop = jnp.arange(N, dtype=jnp.int32)
    src = jnp.concatenate([edge_index[0].astype(jnp.int32), loop])
    dst = jnp.concatenate([edge_index[1].astype(jnp.int32), loop])
    pad_e = E_PAD - src.shape[0]
    src = jnp.concatenate([src, jnp.full((pad_e,), N, jnp.int32)])
    dst = jnp.concatenate([dst, jnp.full((pad_e,), N, jnp.int32)])
    src = src.reshape(NC * NS, CPT, CHUNK)
    dst = dst.reshape(NC * NS, CPT, CHUNK)

    xp = jnp.pad(x, ((0, N_PAD - N), (0, 0)))
    Bs1, Bd1 = _blockdiag(a_src1), _blockdiag(a_dst1)
    Bs2, Bd2 = _blockdiag(a_src2), _blockdiag(a_dst2)
    A_s = jnp.pad(jnp.tile(a_src3.astype(F32).reshape(242, 1), (1, 8)),
                  ((0, 0), (0, 8)))
    A_d = jnp.pad(jnp.tile(a_dst3.astype(F32).reshape(242, 1), (1, 8)),
                  ((0, 0), (0, 8)))
    pv = jnp.concatenate([jnp.zeros((8,), F32),
                          jnp.full((8,), NEGH, F32)]).reshape(1, 16)
    ex8 = jnp.pad(jnp.kron(jnp.eye(8, dtype=F32), jnp.ones((1, 8), F32)),
                  ((0, 8), (0, 0)))
    ex1 = jnp.zeros((16, 64), F32).at[0, :].set(1.0)
    z80 = jnp.zeros((N_PAD, 80), F32)
    W3e = W3.astype(F32)[:, 0::2]
    W3o = W3.astype(F32)[:, 1::2]
    b3e = b3.astype(F32)[0::2].reshape(1, 121)
    b3o = b3.astype(F32)[1::2].reshape(1, 121)
    b1r = b1.astype(F32).reshape(1, 64)
    b2r = b2.astype(F32).reshape(1, 64)

    edge_h8 = _make_edge(True)
    edge_h1 = _make_edge(False)

    big1, ad1, mm1 = _tables1(xp, W1.astype(F32), Bs1, Bd1, pv)
    op1, = edge_h8(src, dst, big1, ad1, mm1, z80)
    big2, ad2, mm2 = _tables_next(op1[0], op1[1], ex8, b1r, W2.astype(F32),
                                  Bs2, Bd2, pv)
    op2, = edge_h8(src, dst, big2, ad2, mm2, z80)
    big3, ad3, mm3 = _tables3(op2[0], op2[1], ex8, b2r, W3.astype(F32),
                              A_s, A_d, pv)
    op3, = edge_h1(src, dst, big3, ad3, mm3, z80)
    s0, s1 = _final(op3[0], op3[1], ex1, b3e, b3o, W3e, W3o)
    return jnp.stack([s0[:N], s1[:N]], axis=-1)
